# baseline (speedup 1.0000x reference)
import functools
import math

import jax
import jax.numpy as jnp
from jax import lax
from jax.experimental import pallas as pl
from jax.experimental.pallas import tpu as pltpu
from jax.experimental.pallas import tpu_sc as plsc

N = 10000
NP = 10240
E = 320000
C = 128
NUM_GRAPHS = 64
GPAD = 72
R_CUTOFF = 6.0
DR = 0.1
SIGMA = R_CUTOFF / 3.0
OUT_DIM = 1800
EEK = 64

_EE_BLK = 3200


def _ee_body(evt_ref, we1_ref, we2_ref, ee1_ref, ee2_ref):
    ev = evt_ref[...]
    l2 = jnp.sum(ev * ev, axis=0, keepdims=True)
    el = jnp.sqrt(l2)
    rg = lax.broadcasted_iota(jnp.int32, (EEK, 1), 0).astype(jnp.float32) * DR
    z = (el - rg) / SIGMA
    ea_t = jnp.exp(-0.5 * z * z) * (1.0 / (SIGMA * math.sqrt(2.0 * math.pi)))
    dn = (((0,), (0,)), ((), ()))
    ee1_ref[...] = lax.dot_general(ea_t, we1_ref[...], dn,
                                   preferred_element_type=jnp.float32)
    ee2_ref[...] = lax.dot_general(ea_t, we2_ref[...], dn,
                                   preferred_element_type=jnp.float32)


def _ee_call(edge_vec_t, we1p, we2p):
    grid = E // _EE_BLK
    return pl.pallas_call(
        _ee_body,
        grid=(grid,),
        in_specs=[
            pl.BlockSpec((3, _EE_BLK), lambda i: (0, i)),
            pl.BlockSpec((EEK, C), lambda i: (0, 0)),
            pl.BlockSpec((EEK, C), lambda i: (0, 0)),
        ],
        out_specs=[
            pl.BlockSpec((_EE_BLK, C), lambda i: (i, 0)),
            pl.BlockSpec((_EE_BLK, C), lambda i: (i, 0)),
        ],
        out_shape=[
            jax.ShapeDtypeStruct((E, C), jnp.float32),
            jax.ShapeDtypeStruct((E, C), jnp.float32),
        ],
    )(edge_vec_t, we1p, we2p)


_XM_BLK = 2048


def _xlr_body(h_ref, wl_ref, bl_ref, wr_ref, br_ref, xl_ref, xr_ref):
    h = h_ref[...]
    xl_ref[...] = jnp.dot(h, wl_ref[...],
                          preferred_element_type=jnp.float32) + bl_ref[...]
    xr_ref[...] = jnp.dot(h, wr_ref[...],
                          preferred_element_type=jnp.float32) + br_ref[...]


def _xlr_call(h, wl, bl, wr, br):
    grid = NP // _XM_BLK
    return pl.pallas_call(
        _xlr_body,
        grid=(grid,),
        in_specs=[
            pl.BlockSpec((_XM_BLK, C), lambda i: (i, 0)),
            pl.BlockSpec((C, C), lambda i: (0, 0)),
            pl.BlockSpec((1, C), lambda i: (0, 0)),
            pl.BlockSpec((C, C), lambda i: (0, 0)),
            pl.BlockSpec((1, C), lambda i: (0, 0)),
        ],
        out_specs=[
            pl.BlockSpec((_XM_BLK, C), lambda i: (i, 0)),
            pl.BlockSpec((_XM_BLK, C), lambda i: (i, 0)),
        ],
        out_shape=[
            jax.ShapeDtypeStruct((NP, C), jnp.float32),
            jax.ShapeDtypeStruct((NP, C), jnp.float32),
        ],
    )(h, wl, bl, wr, br)


_SC_B = 80
_H0 = 48
_H1 = 32
_NSC = 1
_NT = 16 * _NSC
_NHALF = 2
_SC_IT = E // _NHALF // _NT // _SC_B
_CHB = 5
_NCH = _SC_IT // _CHB
_EPT = E // _NHALF // _NT
_ZROWS = NP // 16
_DROWS = NP // C


def _gat_sc_body(half, xl_hbm, xr_hbm, ee_hbm, idx_hbm, att_hbm,
                 out_hbm, den_hbm,
                 idxc, dsts0, dsts1, pb0, pb1, xlv, xrv, eev, msgv, attv,
                 zv, zv1, accum, den_sh,
                 semg0, semg1, sems0, sems1):
    cid = lax.axis_index("c")
    sid = lax.axis_index("s")
    wid = cid * 16 + sid

    def _z(i, _):
        for k in range(C // 16):
            zv[i, pl.ds(k * 16, 16)] = jnp.zeros((16,), jnp.float32)
        return 0
    lax.fori_loop(0, 8, _z, 0)

    def _z1(i, _):
        zv1[pl.ds(i * 16, 16)] = jnp.zeros((16,), jnp.float32)
        return 0
    lax.fori_loop(0, 80, _z1, 0)
    row0 = sid * _ZROWS

    def _zcopy(i, _):
        pltpu.sync_copy(zv, accum.at[pl.ds(row0 + i * 8, 8)])
        return 0
    lax.fori_loop(0, _ZROWS // 8, _zcopy, 0)

    @pl.when(sid == 0)
    def _():
        def _zd(i, _):
            pltpu.sync_copy(zv1, den_sh.at[pl.ds(i * 1280, 1280)])
            return 0
        lax.fori_loop(0, NP // 1280, _zd, 0)

    pltpu.sync_copy(att_hbm, attv)

    plsc.subcore_barrier()

    ebase = (half * _NT + wid) * _EPT
    lane = lax.iota(jnp.int32, 16)
    bfly = tuple(jnp.bitwise_xor(lane, m)[:, None] for m in (8, 4, 2, 1))
    halves = ((0, _H0, dsts0, pb0, semg0, sems0),
              (_H0, _H1, dsts1, pb1, semg1, sems1))

    def _issue_g(b, it, hi):
        h0, hn, dsts, pb, semg, sems = halves[hi]
        e0 = ebase + it * _SC_B + h0
        pltpu.async_copy(xl_hbm.at[idxc.at[b, 0, pl.ds(h0, hn)]],
                         xlv.at[pl.ds(h0, hn)], semg)
        pltpu.async_copy(xr_hbm.at[idxc.at[b, 1, pl.ds(h0, hn)]],
                         xrv.at[pl.ds(h0, hn)], semg)
        pltpu.async_copy(ee_hbm.at[pl.ds(e0, hn)], eev.at[pl.ds(h0, hn)], semg)

    def _wait_g(b, it, hi):
        h0, hn, dsts, pb, semg, sems = halves[hi]
        e0 = ebase + it * _SC_B + h0
        pltpu.make_async_copy(xl_hbm.at[idxc.at[b, 0, pl.ds(h0, hn)]],
                              xlv.at[pl.ds(h0, hn)], semg).wait()
        pltpu.make_async_copy(xr_hbm.at[idxc.at[b, 1, pl.ds(h0, hn)]],
                              xrv.at[pl.ds(h0, hn)], semg).wait()
        pltpu.make_async_copy(ee_hbm.at[pl.ds(e0, hn)],
                              eev.at[pl.ds(h0, hn)], semg).wait()

    def _drain_s(hi):
        h0, hn, dsts, pb, semg, sems = halves[hi]
        pltpu.make_async_copy(msgv.at[pl.ds(h0, hn)], accum.at[dsts],
                              sems).wait()
        pltpu.make_async_copy(pb, den_sh.at[dsts], sems).wait()

    def _compute(b, hi):
        h0, hn, dsts, pb, semg, sems = halves[hi]
        for k in range(hn // 16):
            dsts[pl.ds(k * 16, 16)] = idxc[b, 1, pl.ds(h0 + k * 16, 16)]

        dnums = lax.GatherDimensionNumbers(
            offset_dims=(), collapsed_slice_dims=(0,), start_index_map=(0,))

        def _group(g16, _):
            j0 = h0 + g16 * 16
            pvals = jnp.zeros((16,), jnp.float32)
            for jj in range(16):
                j = j0 + jj
                acc0 = jnp.zeros((16,), jnp.float32)
                acc1 = jnp.zeros((16,), jnp.float32)
                for k in range(C // 16):
                    sv = (xlv[j, pl.ds(k * 16, 16)] + xrv[j, pl.ds(k * 16, 16)]
                          + eev[j, pl.ds(k * 16, 16)])
                    sv = jnp.maximum(sv, 0.2 * sv)
                    if k % 2 == 0:
                        acc0 = acc0 + sv * attv[pl.ds(k * 16, 16)]
                    else:
                        acc1 = acc1 + sv * attv[pl.ds(k * 16, 16)]
                acc = acc0 + acc1
                for bidx in bfly:
                    gth = lax.gather(
                        acc, bidx, dnums, (1,),
                        mode=lax.GatherScatterMode.PROMISE_IN_BOUNDS)
                    acc = acc + gth
                pe = jnp.exp(acc)
                for k in range(C // 16):
                    msgv[j, pl.ds(k * 16, 16)] = pe * xlv[j, pl.ds(k * 16, 16)]
                pvals = jnp.where(lane == jj, pe, pvals)
            pb[pl.ds(g16 * 16, 16)] = pvals
            return 0
        lax.fori_loop(0, hn // 16, _group, 0)

    def _issue_s(hi):
        h0, hn, dsts, pb, semg, sems = halves[hi]
        pltpu.async_copy(msgv.at[pl.ds(h0, hn)], accum.at[dsts], sems,
                         add=True)
        pltpu.async_copy(pb, den_sh.at[dsts], sems, add=True)

    def _chunk(g, _):
        pltpu.sync_copy(idx_hbm.at[half, wid, pl.ds(g * _CHB, _CHB)], idxc)
        it0 = g * _CHB
        _issue_g(0, it0, 0)
        _issue_g(0, it0, 1)

        def _batch(b, _):
            it = it0 + b
            for hi in (0, 1):
                _wait_g(b, it, hi)

                @pl.when(it > 0)
                def _():
                    _drain_s(hi)
                _compute(b, hi)
                _issue_s(hi)

                @pl.when(b < _CHB - 1)
                def _():
                    _issue_g(b + 1, it + 1, hi)
            return 0
        lax.fori_loop(0, _CHB, _batch, 0)
        return 0

    lax.fori_loop(0, _NCH, _chunk, 0)
    _drain_s(0)
    _drain_s(1)

    plsc.subcore_barrier()

    def _drain(i, _):
        pltpu.sync_copy(accum.at[pl.ds(row0 + i * 80, 80)],
                        out_hbm.at[cid, pl.ds(row0 + i * 80, 80)])
        return 0
    lax.fori_loop(0, _ZROWS // 80, _drain, 0)

    @pl.when(sid == 0)
    def _():
        pltpu.sync_copy(den_sh, den_hbm.at[cid])


def _sc_call(xl, xr, ee, idx5d, att, half):
    mesh = plsc.VectorSubcoreMesh(core_axis_name="c", subcore_axis_name="s",
                                  num_cores=_NSC)
    f = pl.kernel(
        functools.partial(_gat_sc_body, half),
        mesh=mesh,
        out_type=[
            jax.ShapeDtypeStruct((_NSC, NP, C), jnp.float32),
            jax.ShapeDtypeStruct((_NSC, NP), jnp.float32),
        ],
        scratch_types=[
            pltpu.VMEM((_CHB, 2, _SC_B), jnp.int32),
            pltpu.VMEM((_H0,), jnp.int32),
            pltpu.VMEM((_H1,), jnp.int32),
            pltpu.VMEM((_H0,), jnp.float32),
            pltpu.VMEM((_H1,), jnp.float32),
            pltpu.VMEM((_SC_B, C), jnp.float32),
            pltpu.VMEM((_SC_B, C), jnp.float32),
            pltpu.VMEM((_SC_B, C), jnp.float32),
            pltpu.VMEM((_SC_B, C), jnp.float32),
            pltpu.VMEM((C,), jnp.float32),
            pltpu.VMEM((8, C), jnp.float32),
            pltpu.VMEM((1280,), jnp.float32),
            pltpu.VMEM_SHARED((NP, C), jnp.float32),
            pltpu.VMEM_SHARED((NP,), jnp.float32),
            pltpu.SemaphoreType.DMA,
            pltpu.SemaphoreType.DMA,
            pltpu.SemaphoreType.DMA,
            pltpu.SemaphoreType.DMA,
        ],
    )
    return f(xl, xr, ee, idx5d, att)


_CB_BLK = 2048


def _combine_body(acca_ref, accb_ref, dena_ref, denb_ref, bias_ref,
                  batch_ref, h_ref, g_ref):
    i = pl.program_id(0)

    @pl.when(i == 0)
    def _():
        g_ref[...] = jnp.full((GPAD, C), -jnp.inf, jnp.float32)

    num = acca_ref[0] + accb_ref[0]
    den = dena_ref[0] + denb_ref[0]
    h = num / (den + 1e-16) + bias_ref[...]
    h = jnp.maximum(h, 0.0)
    h_ref[...] = h

    bcol = batch_ref[0]
    jmin = jnp.min(bcol)
    jmax = jnp.max(bcol)

    def _graph(j, _):
        row = jnp.max(jnp.where(bcol == j, h, -jnp.inf), axis=0,
                      keepdims=True)
        g_ref[pl.ds(j, 1), :] = jnp.maximum(g_ref[pl.ds(j, 1), :], row)
        return 0
    lax.fori_loop(jmin, jmax + 1, _graph, 0)


def _combine_call(acca, accb, dena, denb, bias, batch3d):
    grid = NP // _CB_BLK
    return pl.pallas_call(
        _combine_body,
        grid=(grid,),
        in_specs=[
            pl.BlockSpec((_NSC, _CB_BLK, C), lambda i: (0, i, 0)),
            pl.BlockSpec((_NSC, _CB_BLK, C), lambda i: (0, i, 0)),
            pl.BlockSpec((_NSC, _CB_BLK, 1), lambda i: (0, i, 0)),
            pl.BlockSpec((_NSC, _CB_BLK, 1), lambda i: (0, i, 0)),
            pl.BlockSpec((1, C), lambda i: (0, 0)),
            pl.BlockSpec((1, _CB_BLK, 1), lambda i: (i, 0, 0)),
        ],
        out_specs=[
            pl.BlockSpec((_CB_BLK, C), lambda i: (i, 0)),
            pl.BlockSpec((GPAD, C), lambda i: (0, 0)),
        ],
        out_shape=[
            jax.ShapeDtypeStruct((NP, C), jnp.float32),
            jax.ShapeDtypeStruct((GPAD, C), jnp.float32),
        ],
    )(acca, accb, dena, denb, bias, batch3d)


def _mlp_body(g1_ref, g2_ref, w1_ref, b1_ref, bng_ref, bnb_ref, w2_ref,
              b2_ref, out_ref):
    g = g1_ref[...][:NUM_GRAPHS] + g2_ref[...][:NUM_GRAPHS]
    gf = jnp.dot(g, w1_ref[...], preferred_element_type=jnp.float32) + b1_ref[...]
    inv = 1.0 / math.sqrt(1.0 + 1e-5)
    gf = gf * (inv * bng_ref[...]) + bnb_ref[...]
    gf = jnp.maximum(gf, 0.0)
    out_ref[...] = jnp.dot(gf, w2_ref[...],
                           preferred_element_type=jnp.float32) + b2_ref[...]


def _mlp_call(g1, g2, w1, b1, bng, bnb, w2, b2):
    return pl.pallas_call(
        _mlp_body,
        out_shape=jax.ShapeDtypeStruct((NUM_GRAPHS, OUT_DIM), jnp.float32),
    )(g1, g2, w1, b1, bng, bnb, w2, b2)


def kernel(x, edge_index, edge_vec, batch, params):
    idx5d = jnp.stack(
        [edge_index[0].reshape(_NHALF, _NT, _SC_IT, _SC_B),
         edge_index[1].reshape(_NHALF, _NT, _SC_IT, _SC_B)], axis=3)
    batch3d = jnp.concatenate(
        [batch, jnp.full((NP - N,), NUM_GRAPHS, batch.dtype)]
    ).reshape(NP // _CB_BLK, _CB_BLK, 1)

    p1, p2 = params['conv0'], params['conv1']
    we1p = jnp.concatenate(
        [p1['We'], jnp.zeros((EEK - p1['We'].shape[0], C), jnp.float32)], axis=0)
    we2p = jnp.concatenate(
        [p2['We'], jnp.zeros((EEK - p2['We'].shape[0], C), jnp.float32)], axis=0)
    ee1, ee2 = _ee_call(edge_vec.T, we1p, we2p)

    h = jnp.concatenate([x, jnp.zeros((NP - N, C), jnp.float32)], axis=0)
    gs = []
    for p, ee in ((p1, ee1), (p2, ee2)):
        xl, xr = _xlr_call(h, p['Wl'], p['bl'].reshape(1, C),
                           p['Wr'], p['br'].reshape(1, C))
        acca, dena = _sc_call(xl, xr, ee, idx5d, p['att'].reshape(C), 0)
        accb, denb = _sc_call(xl, xr, ee, idx5d, p['att'].reshape(C), 1)
        h, g = _combine_call(acca, accb, dena.reshape(_NSC, NP, 1),
                             denb.reshape(_NSC, NP, 1),
                             p['bias'].reshape(1, C), batch3d)
        gs.append(g)

    return _mlp_call(gs[0], gs[1],
                     params['lin1_W'], params['lin1_b'].reshape(1, C),
                     params['bn_g'].reshape(1, C), params['bn_b'].reshape(1, C),
                     params['lin2_W'], params['lin2_b'].reshape(1, OUT_DIM))


# speedup vs baseline: 1.0689x; 1.0689x over previous
import functools
import math

import jax
import jax.numpy as jnp
from jax import lax
from jax.experimental import pallas as pl
from jax.experimental.pallas import tpu as pltpu
from jax.experimental.pallas import tpu_sc as plsc

N = 10000
NP = 10240
E = 320000
C = 128
NUM_GRAPHS = 64
GPAD = 72
R_CUTOFF = 6.0
DR = 0.1
SIGMA = R_CUTOFF / 3.0
OUT_DIM = 1800
EEK = 64

_EE_BLK = 3200


def _ee_body(evt_ref, we1_ref, we2_ref, ee1_ref, ee2_ref):
    ev = evt_ref[...]
    l2 = jnp.sum(ev * ev, axis=0, keepdims=True)
    el = jnp.sqrt(l2)
    rg = lax.broadcasted_iota(jnp.int32, (EEK, 1), 0).astype(jnp.float32) * DR
    z = (el - rg) / SIGMA
    ea_t = jnp.exp(-0.5 * z * z) * (1.0 / (SIGMA * math.sqrt(2.0 * math.pi)))
    dn = (((0,), (0,)), ((), ()))
    ee1_ref[...] = lax.dot_general(ea_t, we1_ref[...], dn,
                                   preferred_element_type=jnp.float32)
    ee2_ref[...] = lax.dot_general(ea_t, we2_ref[...], dn,
                                   preferred_element_type=jnp.float32)


def _ee_call(edge_vec_t, we1p, we2p):
    grid = E // _EE_BLK
    return pl.pallas_call(
        _ee_body,
        grid=(grid,),
        in_specs=[
            pl.BlockSpec((3, _EE_BLK), lambda i: (0, i)),
            pl.BlockSpec((EEK, C), lambda i: (0, 0)),
            pl.BlockSpec((EEK, C), lambda i: (0, 0)),
        ],
        out_specs=[
            pl.BlockSpec((_EE_BLK, C), lambda i: (i, 0)),
            pl.BlockSpec((_EE_BLK, C), lambda i: (i, 0)),
        ],
        out_shape=[
            jax.ShapeDtypeStruct((E, C), jnp.float32),
            jax.ShapeDtypeStruct((E, C), jnp.float32),
        ],
    )(edge_vec_t, we1p, we2p)


_XM_BLK = 2048


def _xlr_body(h_ref, wl_ref, bl_ref, wr_ref, br_ref, xl_ref, xr_ref):
    h = h_ref[...]
    xl_ref[...] = jnp.dot(h, wl_ref[...],
                          preferred_element_type=jnp.float32) + bl_ref[...]
    xr_ref[...] = jnp.dot(h, wr_ref[...],
                          preferred_element_type=jnp.float32) + br_ref[...]


def _xlr_call(h, wl, bl, wr, br):
    grid = NP // _XM_BLK
    return pl.pallas_call(
        _xlr_body,
        grid=(grid,),
        in_specs=[
            pl.BlockSpec((_XM_BLK, C), lambda i: (i, 0)),
            pl.BlockSpec((C, C), lambda i: (0, 0)),
            pl.BlockSpec((1, C), lambda i: (0, 0)),
            pl.BlockSpec((C, C), lambda i: (0, 0)),
            pl.BlockSpec((1, C), lambda i: (0, 0)),
        ],
        out_specs=[
            pl.BlockSpec((_XM_BLK, C), lambda i: (i, 0)),
            pl.BlockSpec((_XM_BLK, C), lambda i: (i, 0)),
        ],
        out_shape=[
            jax.ShapeDtypeStruct((NP, C), jnp.float32),
            jax.ShapeDtypeStruct((NP, C), jnp.float32),
        ],
    )(h, wl, bl, wr, br)


_SC_B = 80
_H0 = 48
_H1 = 32
_NSC = 1
_NT = 16 * _NSC
_NHALF = 1
_SC_IT = E // _NHALF // _NT // _SC_B
_CHB = 10
_NCH = _SC_IT // _CHB
_EPT = E // _NHALF // _NT
_ZROWS = NP // 16
_DROWS = NP // C


def _gat_sc_body(half, xl_hbm, xr_hbm, ee_hbm, idx_hbm, att_hbm,
                 out_hbm, den_hbm,
                 idxc, dsts0, dsts1, pb0, pb1, xlv, xrv, eev, msgv, attv,
                 zv, zv1, accum, den_sh,
                 semg0, semg1, sems0, sems1):
    cid = lax.axis_index("c")
    sid = lax.axis_index("s")
    wid = cid * 16 + sid

    def _z(i, _):
        for k in range(C // 16):
            zv[i, pl.ds(k * 16, 16)] = jnp.zeros((16,), jnp.float32)
        return 0
    lax.fori_loop(0, 8, _z, 0)

    def _z1(i, _):
        zv1[pl.ds(i * 16, 16)] = jnp.zeros((16,), jnp.float32)
        return 0
    lax.fori_loop(0, 80, _z1, 0)
    row0 = sid * _ZROWS

    def _zcopy(i, _):
        pltpu.sync_copy(zv, accum.at[pl.ds(row0 + i * 8, 8)])
        return 0
    lax.fori_loop(0, _ZROWS // 8, _zcopy, 0)

    @pl.when(sid == 0)
    def _():
        def _zd(i, _):
            pltpu.sync_copy(zv1, den_sh.at[pl.ds(i * 1280, 1280)])
            return 0
        lax.fori_loop(0, NP // 1280, _zd, 0)

    pltpu.sync_copy(att_hbm, attv)

    plsc.subcore_barrier()

    ebase = (half * _NT + wid) * _EPT
    lane = lax.iota(jnp.int32, 16)
    bfly = tuple(jnp.bitwise_xor(lane, m)[:, None] for m in (8, 4, 2, 1))
    halves = ((0, _H0, dsts0, pb0, semg0, sems0),
              (_H0, _H1, dsts1, pb1, semg1, sems1))

    def _issue_g(b, it, hi):
        h0, hn, dsts, pb, semg, sems = halves[hi]
        e0 = ebase + it * _SC_B + h0
        pltpu.async_copy(xl_hbm.at[idxc.at[b, 0, pl.ds(h0, hn)]],
                         xlv.at[pl.ds(h0, hn)], semg)
        pltpu.async_copy(xr_hbm.at[idxc.at[b, 1, pl.ds(h0, hn)]],
                         xrv.at[pl.ds(h0, hn)], semg)
        pltpu.async_copy(ee_hbm.at[pl.ds(e0, hn)], eev.at[pl.ds(h0, hn)], semg)

    def _wait_g(b, it, hi):
        h0, hn, dsts, pb, semg, sems = halves[hi]
        e0 = ebase + it * _SC_B + h0
        pltpu.make_async_copy(xl_hbm.at[idxc.at[b, 0, pl.ds(h0, hn)]],
                              xlv.at[pl.ds(h0, hn)], semg).wait()
        pltpu.make_async_copy(xr_hbm.at[idxc.at[b, 1, pl.ds(h0, hn)]],
                              xrv.at[pl.ds(h0, hn)], semg).wait()
        pltpu.make_async_copy(ee_hbm.at[pl.ds(e0, hn)],
                              eev.at[pl.ds(h0, hn)], semg).wait()

    def _drain_s(hi):
        h0, hn, dsts, pb, semg, sems = halves[hi]
        pltpu.make_async_copy(msgv.at[pl.ds(h0, hn)], accum.at[dsts],
                              sems).wait()
        pltpu.make_async_copy(pb, den_sh.at[dsts], sems).wait()

    def _compute(b, hi):
        h0, hn, dsts, pb, semg, sems = halves[hi]
        for k in range(hn // 16):
            dsts[pl.ds(k * 16, 16)] = idxc[b, 1, pl.ds(h0 + k * 16, 16)]

        dnums = lax.GatherDimensionNumbers(
            offset_dims=(), collapsed_slice_dims=(0,), start_index_map=(0,))

        def _group(g16, _):
            j0 = h0 + g16 * 16
            pvals = jnp.zeros((16,), jnp.float32)
            for jj in range(16):
                j = j0 + jj
                acc0 = jnp.zeros((16,), jnp.float32)
                acc1 = jnp.zeros((16,), jnp.float32)
                for k in range(C // 16):
                    sv = (xlv[j, pl.ds(k * 16, 16)] + xrv[j, pl.ds(k * 16, 16)]
                          + eev[j, pl.ds(k * 16, 16)])
                    sv = jnp.maximum(sv, 0.2 * sv)
                    if k % 2 == 0:
                        acc0 = acc0 + sv * attv[pl.ds(k * 16, 16)]
                    else:
                        acc1 = acc1 + sv * attv[pl.ds(k * 16, 16)]
                acc = acc0 + acc1
                for bidx in bfly:
                    gth = lax.gather(
                        acc, bidx, dnums, (1,),
                        mode=lax.GatherScatterMode.PROMISE_IN_BOUNDS)
                    acc = acc + gth
                pe = jnp.exp(acc)
                for k in range(C // 16):
                    msgv[j, pl.ds(k * 16, 16)] = pe * xlv[j, pl.ds(k * 16, 16)]
                pvals = jnp.where(lane == jj, pe, pvals)
            pb[pl.ds(g16 * 16, 16)] = pvals
            return 0
        lax.fori_loop(0, hn // 16, _group, 0)

    def _issue_s(hi):
        h0, hn, dsts, pb, semg, sems = halves[hi]
        pltpu.async_copy(msgv.at[pl.ds(h0, hn)], accum.at[dsts], sems,
                         add=True)
        pltpu.async_copy(pb, den_sh.at[dsts], sems, add=True)

    def _chunk(g, _):
        pltpu.sync_copy(idx_hbm.at[half, wid, pl.ds(g * _CHB, _CHB)], idxc)
        it0 = g * _CHB
        _issue_g(0, it0, 0)
        _issue_g(0, it0, 1)

        def _batch(b, _):
            it = it0 + b
            for hi in (0, 1):
                _wait_g(b, it, hi)

                @pl.when(it > 0)
                def _():
                    _drain_s(hi)
                _compute(b, hi)
                _issue_s(hi)

                @pl.when(b < _CHB - 1)
                def _():
                    _issue_g(b + 1, it + 1, hi)
            return 0
        lax.fori_loop(0, _CHB, _batch, 0)
        return 0

    lax.fori_loop(0, _NCH, _chunk, 0)
    _drain_s(0)
    _drain_s(1)

    plsc.subcore_barrier()

    def _drain(i, _):
        pltpu.sync_copy(accum.at[pl.ds(row0 + i * 80, 80)],
                        out_hbm.at[cid, pl.ds(row0 + i * 80, 80)])
        return 0
    lax.fori_loop(0, _ZROWS // 80, _drain, 0)

    @pl.when(sid == 0)
    def _():
        pltpu.sync_copy(den_sh, den_hbm.at[cid])


def _sc_call(xl, xr, ee, idx5d, att, half):
    mesh = plsc.VectorSubcoreMesh(core_axis_name="c", subcore_axis_name="s",
                                  num_cores=_NSC)
    f = pl.kernel(
        functools.partial(_gat_sc_body, half),
        mesh=mesh,
        out_type=[
            jax.ShapeDtypeStruct((_NSC, NP, C), jnp.float32),
            jax.ShapeDtypeStruct((_NSC, NP), jnp.float32),
        ],
        scratch_types=[
            pltpu.VMEM((_CHB, 2, _SC_B), jnp.int32),
            pltpu.VMEM((_H0,), jnp.int32),
            pltpu.VMEM((_H1,), jnp.int32),
            pltpu.VMEM((_H0,), jnp.float32),
            pltpu.VMEM((_H1,), jnp.float32),
            pltpu.VMEM((_SC_B, C), jnp.float32),
            pltpu.VMEM((_SC_B, C), jnp.float32),
            pltpu.VMEM((_SC_B, C), jnp.float32),
            pltpu.VMEM((_SC_B, C), jnp.float32),
            pltpu.VMEM((C,), jnp.float32),
            pltpu.VMEM((8, C), jnp.float32),
            pltpu.VMEM((1280,), jnp.float32),
            pltpu.VMEM_SHARED((NP, C), jnp.float32),
            pltpu.VMEM_SHARED((NP,), jnp.float32),
            pltpu.SemaphoreType.DMA,
            pltpu.SemaphoreType.DMA,
            pltpu.SemaphoreType.DMA,
            pltpu.SemaphoreType.DMA,
        ],
    )
    return f(xl, xr, ee, idx5d, att)


_CB_BLK = 2048


def _combine_body(acca_ref, dena_ref, bias_ref,
                  batch_ref, h_ref, g_ref):
    i = pl.program_id(0)

    @pl.when(i == 0)
    def _():
        g_ref[...] = jnp.full((GPAD, C), -jnp.inf, jnp.float32)

    num = acca_ref[0]
    den = dena_ref[0]
    h = num / (den + 1e-16) + bias_ref[...]
    h = jnp.maximum(h, 0.0)
    h_ref[...] = h

    bcol = batch_ref[0]
    jmin = jnp.min(bcol)
    jmax = jnp.max(bcol)

    def _graph(j, _):
        row = jnp.max(jnp.where(bcol == j, h, -jnp.inf), axis=0,
                      keepdims=True)
        g_ref[pl.ds(j, 1), :] = jnp.maximum(g_ref[pl.ds(j, 1), :], row)
        return 0
    lax.fori_loop(jmin, jmax + 1, _graph, 0)


def _combine_call(acca, dena, bias, batch3d):
    grid = NP // _CB_BLK
    return pl.pallas_call(
        _combine_body,
        grid=(grid,),
        in_specs=[
            pl.BlockSpec((_NSC, _CB_BLK, C), lambda i: (0, i, 0)),
            pl.BlockSpec((_NSC, _CB_BLK, 1), lambda i: (0, i, 0)),
            pl.BlockSpec((1, C), lambda i: (0, 0)),
            pl.BlockSpec((1, _CB_BLK, 1), lambda i: (i, 0, 0)),
        ],
        out_specs=[
            pl.BlockSpec((_CB_BLK, C), lambda i: (i, 0)),
            pl.BlockSpec((GPAD, C), lambda i: (0, 0)),
        ],
        out_shape=[
            jax.ShapeDtypeStruct((NP, C), jnp.float32),
            jax.ShapeDtypeStruct((GPAD, C), jnp.float32),
        ],
    )(acca, dena, bias, batch3d)


def _mlp_body(g1_ref, g2_ref, w1_ref, b1_ref, bng_ref, bnb_ref, w2_ref,
              b2_ref, out_ref):
    g = g1_ref[...][:NUM_GRAPHS] + g2_ref[...][:NUM_GRAPHS]
    gf = jnp.dot(g, w1_ref[...], preferred_element_type=jnp.float32) + b1_ref[...]
    inv = 1.0 / math.sqrt(1.0 + 1e-5)
    gf = gf * (inv * bng_ref[...]) + bnb_ref[...]
    gf = jnp.maximum(gf, 0.0)
    out_ref[...] = jnp.dot(gf, w2_ref[...],
                           preferred_element_type=jnp.float32) + b2_ref[...]


def _mlp_call(g1, g2, w1, b1, bng, bnb, w2, b2):
    return pl.pallas_call(
        _mlp_body,
        out_shape=jax.ShapeDtypeStruct((NUM_GRAPHS, OUT_DIM), jnp.float32),
    )(g1, g2, w1, b1, bng, bnb, w2, b2)


def kernel(x, edge_index, edge_vec, batch, params):
    idx5d = jnp.stack(
        [edge_index[0].reshape(_NHALF, _NT, _SC_IT, _SC_B),
         edge_index[1].reshape(_NHALF, _NT, _SC_IT, _SC_B)], axis=3)
    batch3d = jnp.concatenate(
        [batch, jnp.full((NP - N,), NUM_GRAPHS, batch.dtype)]
    ).reshape(NP // _CB_BLK, _CB_BLK, 1)

    p1, p2 = params['conv0'], params['conv1']
    we1p = jnp.concatenate(
        [p1['We'], jnp.zeros((EEK - p1['We'].shape[0], C), jnp.float32)], axis=0)
    we2p = jnp.concatenate(
        [p2['We'], jnp.zeros((EEK - p2['We'].shape[0], C), jnp.float32)], axis=0)
    ee1, ee2 = _ee_call(edge_vec.T, we1p, we2p)

    h = jnp.concatenate([x, jnp.zeros((NP - N, C), jnp.float32)], axis=0)
    gs = []
    for p, ee in ((p1, ee1), (p2, ee2)):
        xl, xr = _xlr_call(h, p['Wl'], p['bl'].reshape(1, C),
                           p['Wr'], p['br'].reshape(1, C))
        acca, dena = _sc_call(xl, xr, ee, idx5d, p['att'].reshape(C), 0)
        h, g = _combine_call(acca, dena.reshape(_NSC, NP, 1),
                             p['bias'].reshape(1, C), batch3d)
        gs.append(g)

    return _mlp_call(gs[0], gs[1],
                     params['lin1_W'], params['lin1_b'].reshape(1, C),
                     params['bn_g'].reshape(1, C), params['bn_b'].reshape(1, C),
                     params['lin2_W'], params['lin2_b'].reshape(1, OUT_DIM))
